# Initial kernel scaffold; baseline (speedup 1.0000x reference)
import jax
import jax.numpy as jnp
from jax import lax
from jax.experimental import pallas as pl
from jax.experimental.pallas import tpu as pltpu
from jax.experimental.pallas import tpu_sc as plsc

N = 10000
E = 160000
IN_FEAT = 128
OUT_FEAT = 128
NUM_RELS = 4

NC = 2
NS = 16
L = 16
NW = NC * NS

C = 128
NCHUNK = E // C
CHUNKS_PER_W = -(-NCHUNK // NW)
ROWS_PER_TILE = N // NS
ZROWS = 125
NZ = ROWS_PER_TILE // ZROWS

ROW_BLK = 500
NB = N // ROW_BLK


def _dense_body(feat_ref, wm_ref, wf_ref, lw_ref, bias_ref,
                m_ref, f_ref, base_ref):
    x = feat_ref[...]
    m_ref[...] = jnp.dot(x, wm_ref[...], preferred_element_type=jnp.float32)
    f_ref[...] = jnp.dot(x, wf_ref[...], preferred_element_type=jnp.float32)
    base_ref[...] = (
        jnp.dot(x, lw_ref[...], preferred_element_type=jnp.float32)
        + bias_ref[...]
    )


_dense_call = pl.pallas_call(
    _dense_body,
    grid=(NB,),
    in_specs=[
        pl.BlockSpec((ROW_BLK, IN_FEAT), lambda i: (i, 0)),
        pl.BlockSpec((IN_FEAT, NUM_RELS * OUT_FEAT), lambda i: (0, 0)),
        pl.BlockSpec((IN_FEAT, NUM_RELS * 2 * OUT_FEAT), lambda i: (0, 0)),
        pl.BlockSpec((IN_FEAT, OUT_FEAT), lambda i: (0, 0)),
        pl.BlockSpec((1, OUT_FEAT), lambda i: (0, 0)),
    ],
    out_specs=[
        pl.BlockSpec((ROW_BLK, NUM_RELS * OUT_FEAT), lambda i: (i, 0)),
        pl.BlockSpec((ROW_BLK, NUM_RELS * 2 * OUT_FEAT), lambda i: (i, 0)),
        pl.BlockSpec((ROW_BLK, OUT_FEAT), lambda i: (i, 0)),
    ],
    out_shape=[
        jax.ShapeDtypeStruct((N, NUM_RELS * OUT_FEAT), jnp.float32),
        jax.ShapeDtypeStruct((N, NUM_RELS * 2 * OUT_FEAT), jnp.float32),
        jax.ShapeDtypeStruct((N, OUT_FEAT), jnp.float32),
    ],
)


def _edge_body(m_hbm, f_hbm, src_hbm, dst_hbm, et_hbm, out_hbm,
               acc_sh, src_v, dst_v, et_v, idxm_v, idxf_v,
               m_v, f_v, msg_v, stage_v, sem_m, sem_f):
    c = lax.axis_index("c")
    s = lax.axis_index("s")
    wid = s * NC + c

    zero = jnp.zeros((L,), jnp.float32)

    def _zrow(i, _):
        for k in range(OUT_FEAT // L):
            stage_v[i, pl.ds(k * L, L)] = zero
        return 0

    lax.fori_loop(0, ZROWS, _zrow, 0)
    for z in range(NZ):
        r0 = s * ROWS_PER_TILE + z * ZROWS
        pltpu.sync_copy(stage_v, acc_sh.at[pl.ds(r0, ZROWS), :])
    plsc.subcore_barrier()

    def _chunk(k, _):
        ci = k * NW + wid

        @pl.when(ci < NCHUNK)
        def _():
            off = ci * C
            pltpu.sync_copy(src_hbm.at[pl.ds(off, C)], src_v)
            pltpu.sync_copy(dst_hbm.at[pl.ds(off, C)], dst_v)
            pltpu.sync_copy(et_hbm.at[pl.ds(off, C)], et_v)
            for j in range(C // L):
                sl = pl.ds(j * L, L)
                etj = et_v[sl]
                idxm_v[sl] = src_v[sl] * NUM_RELS + etj
                idxf_v[sl] = dst_v[sl] * NUM_RELS + etj
            cp_m = pltpu.async_copy(m_hbm.at[idxm_v], m_v, sem_m)
            cp_f = pltpu.async_copy(f_hbm.at[idxf_v], f_v, sem_f)
            cp_m.wait()
            cp_f.wait()

            def _edge(j, _):
                for kk in range(OUT_FEAT // L):
                    sl = pl.ds(kk * L, L)
                    mm = m_v[j, sl]
                    gg = f_v[j, sl]
                    bb = f_v[j, pl.ds(OUT_FEAT + kk * L, L)]
                    msg_v[j, sl] = jnp.maximum(gg * mm + bb, 0.0)
                return 0

            lax.fori_loop(0, C, _edge, 0)
            pltpu.sync_copy(msg_v, acc_sh.at[dst_v], add=True)

        return 0

    lax.fori_loop(0, CHUNKS_PER_W, _chunk, 0)

    plsc.subcore_barrier()
    for z in range(NZ):
        r0 = s * ROWS_PER_TILE + z * ZROWS
        pltpu.sync_copy(acc_sh.at[pl.ds(r0, ZROWS), :], stage_v)
        pltpu.sync_copy(stage_v, out_hbm.at[c, pl.ds(r0, ZROWS), :])


_edge_call = pl.kernel(
    _edge_body,
    out_type=jax.ShapeDtypeStruct((NC, N, OUT_FEAT), jnp.float32),
    mesh=plsc.VectorSubcoreMesh(
        core_axis_name="c", subcore_axis_name="s",
        num_cores=NC, num_subcores=NS,
    ),
    scratch_types=[
        pltpu.VMEM_SHARED((N, OUT_FEAT), jnp.float32),
        pltpu.VMEM((C,), jnp.int32),
        pltpu.VMEM((C,), jnp.int32),
        pltpu.VMEM((C,), jnp.int32),
        pltpu.VMEM((C,), jnp.int32),
        pltpu.VMEM((C,), jnp.int32),
        pltpu.VMEM((C, OUT_FEAT), jnp.float32),
        pltpu.VMEM((C, 2 * OUT_FEAT), jnp.float32),
        pltpu.VMEM((C, OUT_FEAT), jnp.float32),
        pltpu.VMEM((ZROWS, OUT_FEAT), jnp.float32),
        pltpu.SemaphoreType.DMA,
        pltpu.SemaphoreType.DMA,
    ],
)


def _combine_body(p_ref, base_ref, o_ref):
    o_ref[...] = p_ref[0] + p_ref[1] + base_ref[...]


_combine_call = pl.pallas_call(
    _combine_body,
    grid=(NB,),
    in_specs=[
        pl.BlockSpec((NC, ROW_BLK, OUT_FEAT), lambda i: (0, i, 0)),
        pl.BlockSpec((ROW_BLK, OUT_FEAT), lambda i: (i, 0)),
    ],
    out_specs=pl.BlockSpec((ROW_BLK, OUT_FEAT), lambda i: (i, 0)),
    out_shape=jax.ShapeDtypeStruct((N, OUT_FEAT), jnp.float32),
)


@jax.jit
def _impl(feat, edge_index, etypes, W, film_W, h_bias, loop_weight):
    wm = jnp.transpose(W, (1, 0, 2)).reshape(IN_FEAT, NUM_RELS * OUT_FEAT)
    wf = jnp.transpose(film_W, (1, 0, 2)).reshape(
        IN_FEAT, NUM_RELS * 2 * OUT_FEAT)
    bias2 = h_bias.reshape(1, OUT_FEAT)

    m_tab, f_tab, base = _dense_call(feat, wm, wf, loop_weight, bias2)
    m2 = m_tab.reshape(N * NUM_RELS, OUT_FEAT)
    f2 = f_tab.reshape(N * NUM_RELS, 2 * OUT_FEAT)

    src = edge_index[0].astype(jnp.int32)
    dst = edge_index[1].astype(jnp.int32)
    et = etypes.astype(jnp.int32)

    partial = _edge_call(m2, f2, src, dst, et)
    return _combine_call(partial, base)


def kernel(feat, edge_index, etypes, W, film_W, h_bias, loop_weight):
    return _impl(feat, edge_index, etypes, W, film_W, h_bias, loop_weight)


# speedup vs baseline: 9.8291x; 9.8291x over previous
import jax
import jax.numpy as jnp
from jax import lax
from jax.experimental import pallas as pl
from jax.experimental.pallas import tpu as pltpu
from jax.experimental.pallas import tpu_sc as plsc

N = 10000
E = 160000
IN_FEAT = 128
OUT_FEAT = 128
NUM_RELS = 4

NC = 2
NS = 16
L = 16
NW = NC * NS

C = 64
NCHUNK = E // C
CHUNKS_PER_W = -(-NCHUNK // NW)
N_PAD = 10112
ROWS_PER_TILE = N_PAD // NS
STAGE_CHUNKS = [(z * C, C) for z in range(ROWS_PER_TILE // C)] + [
    ((ROWS_PER_TILE // C) * C, ROWS_PER_TILE % C)]

ROW_BLK = 1000
NB = N // ROW_BLK
DB_BLK = 2000
DNB = N // DB_BLK


def _dense_body(feat_ref, w_ref, fw_ref, m_ref, f_ref):
    x = feat_ref[...]
    m_ref[...] = jnp.dot(x, w_ref[0], preferred_element_type=jnp.float32)
    f_ref[...] = jnp.dot(x, fw_ref[0], preferred_element_type=jnp.float32)


_dense_call = pl.pallas_call(
    _dense_body,
    grid=(NUM_RELS, DNB),
    in_specs=[
        pl.BlockSpec((DB_BLK, IN_FEAT), lambda r, i: (i, 0)),
        pl.BlockSpec((1, IN_FEAT, OUT_FEAT), lambda r, i: (r, 0, 0)),
        pl.BlockSpec((1, IN_FEAT, 2 * OUT_FEAT), lambda r, i: (r, 0, 0)),
    ],
    out_specs=[
        pl.BlockSpec((DB_BLK, OUT_FEAT), lambda r, i: (r * DNB + i, 0)),
        pl.BlockSpec((DB_BLK, 2 * OUT_FEAT), lambda r, i: (r * DNB + i, 0)),
    ],
    out_shape=[
        jax.ShapeDtypeStruct((NUM_RELS * N, OUT_FEAT), jnp.float32),
        jax.ShapeDtypeStruct((NUM_RELS * N, 2 * OUT_FEAT), jnp.float32),
    ],
)


def _edge_body(m_hbm, f_hbm, ei_hbm, et_hbm, out_hbm,
               acc_sh,
               src0, dst0, et0, m0, f0,
               src1, dst1, et1, m1, f1,
               semi0, semm0, semf0, semi1, semm1, semf1):
    c = lax.axis_index("c")
    s = lax.axis_index("s")
    wid = s * NC + c
    row0 = s * ROWS_PER_TILE

    buf = (
        dict(src=src0, dst=dst0, et=et0,
             m=m0, f=f0, semi=semi0, semm=semm0, semf=semf0),
        dict(src=src1, dst=dst1, et=et1,
             m=m1, f=f1, semi=semi1, semm=semm1, semf=semf1),
    )

    zero = jnp.zeros((L,), jnp.float32)

    def _zrow(i, _):
        for k in range(OUT_FEAT // L):
            m0[i, pl.ds(k * L, L)] = zero
        return 0

    lax.fori_loop(0, C, _zrow, 0)
    for off, zr in STAGE_CHUNKS:
        pltpu.sync_copy(m0.at[pl.ds(0, zr), :],
                        acc_sh.at[pl.ds(row0 + off, zr), :])
    plsc.subcore_barrier()

    def _stage_a(k, b):
        ci = k * NW + wid

        @pl.when(ci < NCHUNK)
        def _():
            off = ci * C
            pltpu.async_copy(ei_hbm.at[0, pl.ds(off, C)], b["src"], b["semi"])
            pltpu.async_copy(ei_hbm.at[1, pl.ds(off, C)], b["dst"], b["semi"])
            pltpu.async_copy(et_hbm.at[pl.ds(off, C)], b["et"], b["semi"])

    def _stage_b(k, b):
        ci = k * NW + wid

        @pl.when(ci < NCHUNK)
        def _():
            for r in ("src", "dst", "et"):
                pltpu.make_async_copy(
                    et_hbm.at[pl.ds(0, C)], b[r], b["semi"]).wait()
            for j in range(C // L):
                sl = pl.ds(j * L, L)
                etn = b["et"][sl] * N
                b["src"][sl] = etn + b["src"][sl]
                b["et"][sl] = etn + b["dst"][sl]
            pltpu.async_copy(m_hbm.at[b["src"]], b["m"], b["semm"])
            pltpu.async_copy(f_hbm.at[b["et"]], b["f"], b["semf"])

    def _stage_c(k, b):
        ci = k * NW + wid

        @pl.when(ci < NCHUNK)
        def _():
            pltpu.make_async_copy(
                m_hbm.at[pl.ds(0, C), :], b["m"], b["semm"]).wait()
            pltpu.make_async_copy(
                f_hbm.at[pl.ds(0, C), :], b["f"], b["semf"]).wait()

            @plsc.parallel_loop(0, C, 1, unroll=4)
            def _edge(j):
                for kk in range(OUT_FEAT // L):
                    sl = pl.ds(kk * L, L)
                    mm = b["m"][j, sl]
                    gg = b["f"][j, sl]
                    bb = b["f"][j, pl.ds(OUT_FEAT + kk * L, L)]
                    b["m"][j, sl] = jnp.maximum(gg * mm + bb, 0.0)
            pltpu.sync_copy(b["m"], acc_sh.at[b["dst"]], add=True)

    _stage_a(0, buf[0])
    _stage_a(1, buf[1])
    _stage_b(0, buf[0])

    def _iter(i, _):
        for half in range(2):
            k = 2 * i + half
            b_cur = buf[half]
            b_nxt = buf[1 - half]
            _stage_b(k + 1, b_nxt)
            _stage_c(k, b_cur)
            _stage_a(k + 2, b_cur)
        return 0

    lax.fori_loop(0, (CHUNKS_PER_W + 1) // 2, _iter, 0)

    plsc.subcore_barrier()
    for off, zr in STAGE_CHUNKS:
        pltpu.sync_copy(acc_sh.at[pl.ds(row0 + off, zr), :],
                        m0.at[pl.ds(0, zr), :])
        pltpu.sync_copy(m0.at[pl.ds(0, zr), :],
                        out_hbm.at[c, pl.ds(row0 + off, zr), :])


def _make_edge_call():
    return pl.kernel(
        _edge_body,
        out_type=jax.ShapeDtypeStruct((NC, N_PAD, OUT_FEAT), jnp.float32),
        mesh=plsc.VectorSubcoreMesh(
            core_axis_name="c", subcore_axis_name="s",
            num_cores=NC, num_subcores=NS,
        ),
        scratch_types=(
            [pltpu.VMEM_SHARED((N_PAD, OUT_FEAT), jnp.float32)]
            + 2 * (
                [pltpu.VMEM((C,), jnp.int32)] * 3
                + [pltpu.VMEM((C, OUT_FEAT), jnp.float32),
                   pltpu.VMEM((C, 2 * OUT_FEAT), jnp.float32)]
            )
            + [pltpu.SemaphoreType.DMA] * 6
        ),
    )


def _combine_body(p_ref, feat_ref, lw_ref, bias_ref, o_ref):
    o_ref[...] = (
        p_ref[0] + p_ref[1]
        + jnp.dot(feat_ref[...], lw_ref[...],
                  preferred_element_type=jnp.float32)
        + bias_ref[...]
    )


_combine_call = pl.pallas_call(
    _combine_body,
    grid=(NB,),
    in_specs=[
        pl.BlockSpec((NC, ROW_BLK, OUT_FEAT), lambda i: (0, i, 0)),
        pl.BlockSpec((ROW_BLK, IN_FEAT), lambda i: (i, 0)),
        pl.BlockSpec((IN_FEAT, OUT_FEAT), lambda i: (0, 0)),
        pl.BlockSpec((1, OUT_FEAT), lambda i: (0, 0)),
    ],
    out_specs=pl.BlockSpec((ROW_BLK, OUT_FEAT), lambda i: (i, 0)),
    out_shape=jax.ShapeDtypeStruct((N, OUT_FEAT), jnp.float32),
)


@jax.jit
def _impl(feat, edge_index, etypes, W, film_W, h_bias, loop_weight):
    bias2 = h_bias.reshape(1, OUT_FEAT)

    m2, f2 = _dense_call(feat, W, film_W)

    partial = _make_edge_call()(m2, f2, edge_index, etypes)
    return _combine_call(partial, feat, loop_weight, bias2)


def kernel(feat, edge_index, etypes, W, film_W, h_bias, loop_weight):
    return _impl(feat, edge_index, etypes, W, film_W, h_bias, loop_weight)


# speedup vs baseline: 10.2653x; 1.0444x over previous
import jax
import jax.numpy as jnp
from jax import lax
from jax.experimental import pallas as pl
from jax.experimental.pallas import tpu as pltpu
from jax.experimental.pallas import tpu_sc as plsc

N = 10000
E = 160000
IN_FEAT = 128
OUT_FEAT = 128
NUM_RELS = 4

NC = 2
NS = 16
L = 16
NW = NC * NS

C = 64
NCHUNK = E // C
CHUNKS_PER_W = -(-NCHUNK // NW)
N_PAD = 10112
ROWS_PER_TILE = N_PAD // NS
STAGE_CHUNKS = [(z * C, C) for z in range(ROWS_PER_TILE // C)] + [
    ((ROWS_PER_TILE // C) * C, ROWS_PER_TILE % C)]

ROW_BLK = 1000
NB = N // ROW_BLK
DB_BLK = 2000
DNB = N // DB_BLK


def _dense_body(feat_ref, w_ref, fw_ref, m_ref, f_ref):
    x = feat_ref[...]
    for r in range(NUM_RELS):
        m_ref[r] = jnp.dot(x, w_ref[r], preferred_element_type=jnp.float32)
        f_ref[r] = jnp.dot(x, fw_ref[r], preferred_element_type=jnp.float32)


_dense_call = pl.pallas_call(
    _dense_body,
    grid=(DNB,),
    in_specs=[
        pl.BlockSpec((DB_BLK, IN_FEAT), lambda i: (i, 0)),
        pl.BlockSpec((NUM_RELS, IN_FEAT, OUT_FEAT), lambda i: (0, 0, 0)),
        pl.BlockSpec((NUM_RELS, IN_FEAT, 2 * OUT_FEAT), lambda i: (0, 0, 0)),
    ],
    out_specs=[
        pl.BlockSpec((NUM_RELS, DB_BLK, OUT_FEAT), lambda i: (0, i, 0)),
        pl.BlockSpec((NUM_RELS, DB_BLK, 2 * OUT_FEAT), lambda i: (0, i, 0)),
    ],
    out_shape=[
        jax.ShapeDtypeStruct((NUM_RELS, N, OUT_FEAT), jnp.float32),
        jax.ShapeDtypeStruct((NUM_RELS, N, 2 * OUT_FEAT), jnp.float32),
    ],
)


def _edge_body(m_hbm, f_hbm, ei_hbm, et_hbm, out_hbm,
               acc_sh,
               src0, dst0, et0, m0, f0,
               src1, dst1, et1, m1, f1,
               semi0, semm0, semf0, semi1, semm1, semf1):
    c = lax.axis_index("c")
    s = lax.axis_index("s")
    wid = s * NC + c
    row0 = s * ROWS_PER_TILE

    buf = (
        dict(src=src0, dst=dst0, et=et0,
             m=m0, f=f0, semi=semi0, semm=semm0, semf=semf0),
        dict(src=src1, dst=dst1, et=et1,
             m=m1, f=f1, semi=semi1, semm=semm1, semf=semf1),
    )

    zero = jnp.zeros((L,), jnp.float32)

    def _zrow(i, _):
        for k in range(OUT_FEAT // L):
            m0[i, pl.ds(k * L, L)] = zero
        return 0

    lax.fori_loop(0, C, _zrow, 0)
    for off, zr in STAGE_CHUNKS:
        pltpu.sync_copy(m0.at[pl.ds(0, zr), :],
                        acc_sh.at[pl.ds(row0 + off, zr), :])
    plsc.subcore_barrier()

    def _stage_a(k, b):
        ci = k * NW + wid

        @pl.when(ci < NCHUNK)
        def _():
            off = ci * C
            pltpu.async_copy(ei_hbm.at[0, pl.ds(off, C)], b["src"], b["semi"])
            pltpu.async_copy(ei_hbm.at[1, pl.ds(off, C)], b["dst"], b["semi"])
            pltpu.async_copy(et_hbm.at[pl.ds(off, C)], b["et"], b["semi"])

    def _stage_b(k, b):
        ci = k * NW + wid

        @pl.when(ci < NCHUNK)
        def _():
            for r in ("src", "dst", "et"):
                pltpu.make_async_copy(
                    et_hbm.at[pl.ds(0, C)], b[r], b["semi"]).wait()
            for j in range(C // L):
                sl = pl.ds(j * L, L)
                etn = b["et"][sl] * N
                b["src"][sl] = etn + b["src"][sl]
                b["et"][sl] = etn + b["dst"][sl]
            pltpu.async_copy(m_hbm.at[b["src"]], b["m"], b["semm"])
            pltpu.async_copy(f_hbm.at[b["et"]], b["f"], b["semf"])

    def _stage_c(k, b):
        ci = k * NW + wid

        @pl.when(ci < NCHUNK)
        def _():
            pltpu.make_async_copy(
                m_hbm.at[pl.ds(0, C), :], b["m"], b["semm"]).wait()
            pltpu.make_async_copy(
                f_hbm.at[pl.ds(0, C), :], b["f"], b["semf"]).wait()

            @plsc.parallel_loop(0, C, 1, unroll=4)
            def _edge(j):
                for kk in range(OUT_FEAT // L):
                    sl = pl.ds(kk * L, L)
                    mm = b["m"][j, sl]
                    gg = b["f"][j, sl]
                    bb = b["f"][j, pl.ds(OUT_FEAT + kk * L, L)]
                    b["m"][j, sl] = jnp.maximum(gg * mm + bb, 0.0)
            pltpu.sync_copy(b["m"], acc_sh.at[b["dst"]], add=True)

    _stage_a(0, buf[0])
    _stage_a(1, buf[1])
    _stage_b(0, buf[0])

    def _iter(i, _):
        for half in range(2):
            k = 2 * i + half
            b_cur = buf[half]
            b_nxt = buf[1 - half]
            _stage_b(k + 1, b_nxt)
            _stage_c(k, b_cur)
            _stage_a(k + 2, b_cur)
        return 0

    lax.fori_loop(0, (CHUNKS_PER_W + 1) // 2, _iter, 0)

    plsc.subcore_barrier()
    for off, zr in STAGE_CHUNKS:
        pltpu.sync_copy(acc_sh.at[pl.ds(row0 + off, zr), :],
                        m0.at[pl.ds(0, zr), :])
        pltpu.sync_copy(m0.at[pl.ds(0, zr), :],
                        out_hbm.at[c, pl.ds(row0 + off, zr), :])


def _make_edge_call():
    return pl.kernel(
        _edge_body,
        out_type=jax.ShapeDtypeStruct((NC, N_PAD, OUT_FEAT), jnp.float32),
        mesh=plsc.VectorSubcoreMesh(
            core_axis_name="c", subcore_axis_name="s",
            num_cores=NC, num_subcores=NS,
        ),
        scratch_types=(
            [pltpu.VMEM_SHARED((N_PAD, OUT_FEAT), jnp.float32)]
            + 2 * (
                [pltpu.VMEM((C,), jnp.int32)] * 3
                + [pltpu.VMEM((C, OUT_FEAT), jnp.float32),
                   pltpu.VMEM((C, 2 * OUT_FEAT), jnp.float32)]
            )
            + [pltpu.SemaphoreType.DMA] * 6
        ),
    )


def _combine_body(p_ref, feat_ref, lw_ref, bias_ref, o_ref):
    o_ref[...] = (
        p_ref[0] + p_ref[1]
        + jnp.dot(feat_ref[...], lw_ref[...],
                  preferred_element_type=jnp.float32)
        + bias_ref[...]
    )


_combine_call = pl.pallas_call(
    _combine_body,
    grid=(NB,),
    in_specs=[
        pl.BlockSpec((NC, ROW_BLK, OUT_FEAT), lambda i: (0, i, 0)),
        pl.BlockSpec((ROW_BLK, IN_FEAT), lambda i: (i, 0)),
        pl.BlockSpec((IN_FEAT, OUT_FEAT), lambda i: (0, 0)),
        pl.BlockSpec((1, OUT_FEAT), lambda i: (0, 0)),
    ],
    out_specs=pl.BlockSpec((ROW_BLK, OUT_FEAT), lambda i: (i, 0)),
    out_shape=jax.ShapeDtypeStruct((N, OUT_FEAT), jnp.float32),
)


@jax.jit
def _impl(feat, edge_index, etypes, W, film_W, h_bias, loop_weight):
    bias2 = h_bias.reshape(1, OUT_FEAT)

    m2, f2 = _dense_call(feat, W, film_W)

    partial = _make_edge_call()(
        m2.reshape(NUM_RELS * N, OUT_FEAT),
        f2.reshape(NUM_RELS * N, 2 * OUT_FEAT),
        edge_index, etypes)
    return _combine_call(partial, feat, loop_weight, bias2)


def kernel(feat, edge_index, etypes, W, film_W, h_bias, loop_weight):
    return _impl(feat, edge_index, etypes, W, film_W, h_bias, loop_weight)


# speedup vs baseline: 11.0971x; 1.0810x over previous
import jax
import jax.numpy as jnp
from jax import lax
from jax.experimental import pallas as pl
from jax.experimental.pallas import tpu as pltpu
from jax.experimental.pallas import tpu_sc as plsc

N = 10000
E = 160000
IN_FEAT = 128
OUT_FEAT = 128
NUM_RELS = 4

NC = 2
NS = 16
L = 16
NW = NC * NS

C = 64
NCHUNK = E // C
CHUNKS_PER_W = -(-NCHUNK // NW)
N_PAD = 10112
ROWS_PER_TILE = N_PAD // NS
STAGE_CHUNKS = [(z * C, C) for z in range(ROWS_PER_TILE // C)] + [
    ((ROWS_PER_TILE // C) * C, ROWS_PER_TILE % C)]

ROW_BLK = 1000
NB = N // ROW_BLK
DB_BLK = 2000
DNB = N // DB_BLK


def _dense_body(feat_ref, w_ref, fw_ref, m_ref, f_ref):
    x = feat_ref[...]
    for r in range(NUM_RELS):
        m_ref[r] = jnp.dot(x, w_ref[r], preferred_element_type=jnp.float32)
        f_ref[r] = jnp.dot(x, fw_ref[r], preferred_element_type=jnp.float32)


_dense_call = pl.pallas_call(
    _dense_body,
    grid=(DNB,),
    in_specs=[
        pl.BlockSpec((DB_BLK, IN_FEAT), lambda i: (i, 0)),
        pl.BlockSpec((NUM_RELS, IN_FEAT, OUT_FEAT), lambda i: (0, 0, 0)),
        pl.BlockSpec((NUM_RELS, IN_FEAT, 2 * OUT_FEAT), lambda i: (0, 0, 0)),
    ],
    out_specs=[
        pl.BlockSpec((NUM_RELS, DB_BLK, OUT_FEAT), lambda i: (0, i, 0)),
        pl.BlockSpec((NUM_RELS, DB_BLK, 2 * OUT_FEAT), lambda i: (0, i, 0)),
    ],
    out_shape=[
        jax.ShapeDtypeStruct((NUM_RELS, N, OUT_FEAT), jnp.float32),
        jax.ShapeDtypeStruct((NUM_RELS, N, 2 * OUT_FEAT), jnp.float32),
    ],
)


def _edge_body(m_hbm, f_hbm, ei_hbm, et_hbm, out_hbm,
               acc_sh,
               src0, dst0, et0, m0, f0,
               src1, dst1, et1, m1, f1,
               dsc0, dsc1,
               semi0, semm0, semf0, semi1, semm1, semf1,
               sems0, sems1):
    c = lax.axis_index("c")
    s = lax.axis_index("s")
    wid = s * NC + c
    row0 = s * ROWS_PER_TILE

    buf = (
        dict(src=src0, dst=dst0, et=et0, dsc=dsc0,
             m=m0, f=f0, semi=semi0, semm=semm0, semf=semf0, sems=sems0),
        dict(src=src1, dst=dst1, et=et1, dsc=dsc1,
             m=m1, f=f1, semi=semi1, semm=semm1, semf=semf1, sems=sems1),
    )

    zero = jnp.zeros((L,), jnp.float32)


    def _stage_a(k, b):
        ci = k * NW + wid

        @pl.when(ci < NCHUNK)
        def _():
            off = ci * C
            pltpu.async_copy(ei_hbm.at[0, pl.ds(off, C)], b["src"], b["semi"])
            pltpu.async_copy(ei_hbm.at[1, pl.ds(off, C)], b["dst"], b["semi"])
            pltpu.async_copy(et_hbm.at[pl.ds(off, C)], b["et"], b["semi"])

    def _stage_b(k, b):
        ci = k * NW + wid

        @pl.when(jnp.logical_and(ci < NCHUNK, k >= 2))
        def _():
            pltpu.make_async_copy(
                b["m"], acc_sh.at[b["dsc"]], b["sems"]).wait()

        @pl.when(ci < NCHUNK)
        def _():
            for r in ("src", "dst", "et"):
                pltpu.make_async_copy(
                    et_hbm.at[pl.ds(0, C)], b[r], b["semi"]).wait()
            for j in range(C // L):
                sl = pl.ds(j * L, L)
                etn = b["et"][sl] * N
                b["src"][sl] = etn + b["src"][sl]
                b["et"][sl] = etn + b["dst"][sl]
            pltpu.async_copy(m_hbm.at[b["src"]], b["m"], b["semm"])
            pltpu.async_copy(f_hbm.at[b["et"]], b["f"], b["semf"])

    def _stage_c(k, b):
        ci = k * NW + wid

        @pl.when(ci < NCHUNK)
        def _():
            pltpu.make_async_copy(
                m_hbm.at[pl.ds(0, C), :], b["m"], b["semm"]).wait()
            pltpu.make_async_copy(
                f_hbm.at[pl.ds(0, C), :], b["f"], b["semf"]).wait()

            @plsc.parallel_loop(0, C, 1, unroll=4)
            def _edge(j):
                for kk in range(OUT_FEAT // L):
                    sl = pl.ds(kk * L, L)
                    mm = b["m"][j, sl]
                    gg = b["f"][j, sl]
                    bb = b["f"][j, pl.ds(OUT_FEAT + kk * L, L)]
                    b["m"][j, sl] = jnp.maximum(gg * mm + bb, 0.0)
            for j in range(C // L):
                sl = pl.ds(j * L, L)
                b["dsc"][sl] = b["dst"][sl]
            pltpu.async_copy(b["m"], acc_sh.at[b["dsc"]], b["sems"],
                             add=True)

    _stage_a(0, buf[0])
    _stage_a(1, buf[1])
    _stage_b(0, buf[0])

    def _zrow2(i, _):
        for k in range(OUT_FEAT // L):
            m1[i, pl.ds(k * L, L)] = zero
        return 0

    lax.fori_loop(0, C, _zrow2, 0)
    for off, zr in STAGE_CHUNKS:
        pltpu.sync_copy(m1.at[pl.ds(0, zr), :],
                        acc_sh.at[pl.ds(row0 + off, zr), :])
    plsc.subcore_barrier()

    def _iter(i, _):
        for half in range(2):
            k = 2 * i + half
            b_cur = buf[half]
            b_nxt = buf[1 - half]
            _stage_b(k + 1, b_nxt)
            _stage_c(k, b_cur)
            _stage_a(k + 2, b_cur)
        return 0

    lax.fori_loop(0, (CHUNKS_PER_W + 1) // 2, _iter, 0)

    for b in buf:
        pltpu.make_async_copy(b["m"], acc_sh.at[b["dsc"]], b["sems"]).wait()

    plsc.subcore_barrier()
    for off, zr in STAGE_CHUNKS:
        pltpu.sync_copy(acc_sh.at[pl.ds(row0 + off, zr), :],
                        m0.at[pl.ds(0, zr), :])
        pltpu.sync_copy(m0.at[pl.ds(0, zr), :],
                        out_hbm.at[c, pl.ds(row0 + off, zr), :])


def _make_edge_call():
    return pl.kernel(
        _edge_body,
        out_type=jax.ShapeDtypeStruct((NC, N_PAD, OUT_FEAT), jnp.float32),
        mesh=plsc.VectorSubcoreMesh(
            core_axis_name="c", subcore_axis_name="s",
            num_cores=NC, num_subcores=NS,
        ),
        scratch_types=(
            [pltpu.VMEM_SHARED((N_PAD, OUT_FEAT), jnp.float32)]
            + 2 * (
                [pltpu.VMEM((C,), jnp.int32)] * 3
                + [pltpu.VMEM((C, OUT_FEAT), jnp.float32),
                   pltpu.VMEM((C, 2 * OUT_FEAT), jnp.float32)]
            )
            + [pltpu.VMEM((C,), jnp.int32)] * 2
            + [pltpu.SemaphoreType.DMA] * 8
        ),
    )


def _combine_body(p_ref, feat_ref, lw_ref, bias_ref, o_ref):
    o_ref[...] = (
        p_ref[0] + p_ref[1]
        + jnp.dot(feat_ref[...], lw_ref[...],
                  preferred_element_type=jnp.float32)
        + bias_ref[...]
    )


_combine_call = pl.pallas_call(
    _combine_body,
    grid=(NB,),
    in_specs=[
        pl.BlockSpec((NC, ROW_BLK, OUT_FEAT), lambda i: (0, i, 0)),
        pl.BlockSpec((ROW_BLK, IN_FEAT), lambda i: (i, 0)),
        pl.BlockSpec((IN_FEAT, OUT_FEAT), lambda i: (0, 0)),
        pl.BlockSpec((1, OUT_FEAT), lambda i: (0, 0)),
    ],
    out_specs=pl.BlockSpec((ROW_BLK, OUT_FEAT), lambda i: (i, 0)),
    out_shape=jax.ShapeDtypeStruct((N, OUT_FEAT), jnp.float32),
)


@jax.jit
def _impl(feat, edge_index, etypes, W, film_W, h_bias, loop_weight):
    bias2 = h_bias.reshape(1, OUT_FEAT)

    m2, f2 = _dense_call(feat, W, film_W)

    partial = _make_edge_call()(
        m2.reshape(NUM_RELS * N, OUT_FEAT),
        f2.reshape(NUM_RELS * N, 2 * OUT_FEAT),
        edge_index, etypes)
    return _combine_call(partial, feat, loop_weight, bias2)


def kernel(feat, edge_index, etypes, W, film_W, h_bias, loop_weight):
    return _impl(feat, edge_index, etypes, W, film_W, h_bias, loop_weight)
